# baseline (speedup 1.0000x reference)
import functools

import jax
import jax.numpy as jnp
from jax import lax
from jax.experimental import pallas as pl
from jax.experimental.pallas import tpu as pltpu
from jax.experimental.pallas import tpu_sc as plsc

B, H, W, C = 2, 384, 384, 96
HW = H * W
N = B * HW

NC, NS, LANES = 2, 16, 16
NW = NC * NS

K = 48
QGROUPS = K // LANES
CGROUPS = C // LANES
CHUNKS_PER_ROW = W // K
ROWS_PER_WORKER = (B * H) // NW
CPW = ROWS_PER_WORKER * CHUNKS_PER_ROW
FLOWN = CPW * K
CP = 128
NB = 3
CSHIFT = 3


def _warp_body(table, fy, fx, out, *refs):
    bufs = tuple(refs[i * 15:(i + 1) * 15] for i in range(NB))
    cid = lax.axis_index("c")
    sid = lax.axis_index("s")
    wid = sid * NC + cid
    qoff = wid * FLOWN

    lane = lax.iota(jnp.int32, LANES)

    def compute_and_fire(c, buf):
        fyv, fxv, axv, ayv, itl, itr, ibl, ibr, tlv, trv, blv, brv = \
            buf[:12]
        gsem = buf[13]
        r = wid * ROWS_PER_WORKER + lax.shift_right_logical(c, CSHIFT)
        eighth = lax.bitwise_and(c, CHUNKS_PER_ROW - 1)
        b = jnp.where(r >= H, 1, 0).astype(jnp.int32)
        i = r - b * H
        jbase = eighth * K
        pltpu.sync_copy(fy.at[pl.ds(qoff + c * K, K)], fyv)
        pltpu.sync_copy(fx.at[pl.ds(qoff + c * K, K)], fxv)
        i_f = jnp.broadcast_to(i, (LANES,)).astype(jnp.float32)
        b_off = jnp.broadcast_to(b * HW, (LANES,))
        for g in range(QGROUPS):
            sl = pl.ds(g * LANES, LANES)
            qx = fxv[sl] + (
                jbase + g * LANES + lane).astype(jnp.float32)
            qy = fyv[sl] + i_f
            x0 = jnp.clip(qx.astype(jnp.int32), 0, W - 2)
            y0 = jnp.clip(qy.astype(jnp.int32), 0, H - 2)
            axv[sl] = jnp.clip(qx - x0.astype(jnp.float32), 0.0, 1.0)
            ayv[sl] = jnp.clip(qy - y0.astype(jnp.float32), 0.0, 1.0)
            base = b_off + y0 * W + x0
            itl[sl] = base
            itr[sl] = base + 1
            ibl[sl] = base + W
            ibr[sl] = base + W + 1
        pltpu.async_copy(table.at[itl], tlv, gsem)
        pltpu.async_copy(table.at[itr], trv, gsem)
        pltpu.async_copy(table.at[ibl], blv, gsem)
        pltpu.async_copy(table.at[ibr], brv, gsem)

    def combine(buf):
        axv, ayv = buf[2], buf[3]
        tlv, trv, blv, brv, outv = buf[8:13]

        def q_body(qg, carry):
            qb = qg * LANES
            ax16 = axv[pl.ds(qb, LANES)]
            ay16 = ayv[pl.ds(qb, LANES)]
            for l in range(LANES):
                a_x = ax16[l]
                a_y = ay16[l]
                q = qb + l
                for g in range(CGROUPS):
                    sl = pl.ds(g * LANES, LANES)
                    tl = tlv[q, sl]
                    tr = trv[q, sl]
                    bl = blv[q, sl]
                    br = brv[q, sl]
                    top = a_x * (tr - tl) + tl
                    bot = a_x * (br - bl) + bl
                    outv[q, sl] = a_y * (bot - top) + top
            return carry

        lax.fori_loop(0, QGROUPS, q_body, 0)

    for s in range(NB - 1):
        compute_and_fire(jnp.int32(s), bufs[s])

    def ring_body(t, carry):
        for s in range(NB):
            buf = bufs[s]
            itl, tlv, trv, blv, brv, outv = (
                buf[4], buf[8], buf[9], buf[10], buf[11], buf[12])
            gsem, ssem = buf[13], buf[14]
            c = NB * t + s
            pltpu.make_async_copy(table.at[itl], tlv, gsem).wait()
            pltpu.make_async_copy(table.at[itl], trv, gsem).wait()
            pltpu.make_async_copy(table.at[itl], blv, gsem).wait()
            pltpu.make_async_copy(table.at[itl], brv, gsem).wait()

            nxt = bufs[(s + NB - 1) % NB]

            @pl.when(c + NB - 1 < CPW)
            def _():
                compute_and_fire(c + NB - 1, nxt)

            @pl.when(t >= 1)
            def _():
                pltpu.make_async_copy(outv, out.at[pl.ds(0, K)], ssem).wait()

            combine(buf)
            pltpu.async_copy(outv, out.at[pl.ds(qoff + c * K, K)], ssem)
        return carry

    lax.fori_loop(0, CPW // NB, ring_body, 0)

    for s in range(NB):
        pltpu.make_async_copy(bufs[s][12], out.at[pl.ds(0, K)],
                              bufs[s][14]).wait()


@functools.partial(jax.jit, donate_argnums=())
def _warp(xp, fy, fx):
    mesh = plsc.VectorSubcoreMesh(
        core_axis_name="c", subcore_axis_name="s",
        num_cores=NC, num_subcores=NS)
    kc = jnp.float32
    ki = jnp.int32
    per_buf = [
        pltpu.VMEM((K,), kc),
        pltpu.VMEM((K,), kc),
        pltpu.VMEM((K,), kc),
        pltpu.VMEM((K,), kc),
        pltpu.VMEM((K,), ki),
        pltpu.VMEM((K,), ki),
        pltpu.VMEM((K,), ki),
        pltpu.VMEM((K,), ki),
        pltpu.VMEM((K, CP), kc),
        pltpu.VMEM((K, CP), kc),
        pltpu.VMEM((K, CP), kc),
        pltpu.VMEM((K, CP), kc),
        pltpu.VMEM((K, CP), kc),
        pltpu.SemaphoreType.DMA,
        pltpu.SemaphoreType.DMA,
    ]
    return pl.kernel(
        _warp_body,
        out_type=jax.ShapeDtypeStruct((N, CP), kc),
        mesh=mesh,
        scratch_types=per_buf * NB,
        compiler_params=pltpu.CompilerParams(use_tc_tiling_on_sc=False),
    )(xp, fy, fx)


def kernel(x, flow):
    xp = jnp.pad(x, ((0, 0), (0, 0), (0, 0), (0, CP - C))).reshape(N, CP)
    fy = flow[..., 0].reshape(N)
    fx = flow[..., 1].reshape(N)
    outp = _warp(xp, fy, fx)
    return outp[:, :C].reshape(B, H, W, C)


# speedup vs baseline: 1.1628x; 1.1628x over previous
import functools

import jax
import jax.numpy as jnp
from jax import lax
from jax.experimental import pallas as pl
from jax.experimental.pallas import tpu as pltpu
from jax.experimental.pallas import tpu_sc as plsc

B, H, W, C = 2, 384, 384, 96
HW = H * W
N = B * HW

NC, NS, LANES = 2, 16, 16
NW = NC * NS

K = 96
GROUPS = K // LANES
CHUNKS_PER_ROW = W // K
ROWS_PER_WORKER = (B * H) // NW
CPW = ROWS_PER_WORKER * CHUNKS_PER_ROW
FLOWN = CPW * K
CP = 128


def _warp_body(table, fy, fx, out,
               fyv, fxv,
               ax0, ay0, ax1, ay1,
               itl0, itr0, ibl0, ibr0,
               itl1, itr1, ibl1, ibr1,
               tl0, tr0, bl0, br0,
               tl1, tr1, bl1, br1,
               out0, out1,
               gsem0, gsem1, ssem0, ssem1):
    cid = lax.axis_index("c")
    sid = lax.axis_index("s")
    wid = sid * NC + cid
    qoff = wid * FLOWN

    lane = lax.iota(jnp.int32, LANES)

    bufs = (
        (ax0, ay0, itl0, itr0, ibl0, ibr0, tl0, tr0, bl0, br0, out0,
         gsem0, ssem0),
        (ax1, ay1, itl1, itr1, ibl1, ibr1, tl1, tr1, bl1, br1, out1,
         gsem1, ssem1),
    )

    def compute_and_fire(c, buf):
        axv, ayv, itl, itr, ibl, ibr, tlv, trv, blv, brv = buf[:10]
        gsem = buf[11]
        r = wid * ROWS_PER_WORKER + lax.shift_right_logical(c, 2)
        quarter = lax.bitwise_and(c, CHUNKS_PER_ROW - 1)
        b = jnp.where(r >= H, 1, 0).astype(jnp.int32)
        i = r - b * H
        jbase = quarter * K
        pltpu.sync_copy(fy.at[pl.ds(qoff + c * K, K)], fyv)
        pltpu.sync_copy(fx.at[pl.ds(qoff + c * K, K)], fxv)
        i_f = jnp.broadcast_to(i, (LANES,)).astype(jnp.float32)
        b_off = jnp.broadcast_to(b * HW, (LANES,))
        for g in range(GROUPS):
            sl = pl.ds(g * LANES, LANES)
            qx = fxv[sl] + (
                jbase + g * LANES + lane).astype(jnp.float32)
            qy = fyv[sl] + i_f
            x0 = jnp.clip(qx.astype(jnp.int32), 0, W - 2)
            y0 = jnp.clip(qy.astype(jnp.int32), 0, H - 2)
            axv[sl] = jnp.clip(qx - x0.astype(jnp.float32), 0.0, 1.0)
            ayv[sl] = jnp.clip(qy - y0.astype(jnp.float32), 0.0, 1.0)
            base = b_off + y0 * W + x0
            itl[sl] = base
            itr[sl] = base + 1
            ibl[sl] = base + W
            ibr[sl] = base + W + 1
        pltpu.async_copy(table.at[itl], tlv, gsem)
        pltpu.async_copy(table.at[itr], trv, gsem)
        pltpu.async_copy(table.at[ibl], blv, gsem)
        pltpu.async_copy(table.at[ibr], brv, gsem)

    def combine(buf):
        axv, ayv = buf[0], buf[1]
        tlv, trv, blv, brv, outv = buf[6:11]

        def q_body(qg, carry):
            qb = qg * LANES
            ax16 = axv[pl.ds(qb, LANES)]
            ay16 = ayv[pl.ds(qb, LANES)]
            for l in range(LANES):
                a_x = ax16[l]
                a_y = ay16[l]
                q = qb + l
                for g in range(GROUPS):
                    sl = pl.ds(g * LANES, LANES)
                    tl = tlv[q, sl]
                    tr = trv[q, sl]
                    bl = blv[q, sl]
                    br = brv[q, sl]
                    top = a_x * (tr - tl) + tl
                    bot = a_x * (br - bl) + bl
                    outv[q, sl] = a_y * (bot - top) + top
            return carry

        lax.fori_loop(0, GROUPS, q_body, 0)

    compute_and_fire(jnp.int32(0), bufs[0])
    compute_and_fire(jnp.int32(1), bufs[1])

    def pair_body(t, carry):
        for s in range(2):
            buf = bufs[s]
            itl, tlv, trv, blv, brv, outv = (
                buf[2], buf[6], buf[7], buf[8], buf[9], buf[10])
            gsem, ssem = buf[11], buf[12]
            c = 2 * t + s
            pltpu.make_async_copy(table.at[itl], tlv, gsem).wait()
            pltpu.make_async_copy(table.at[itl], trv, gsem).wait()
            pltpu.make_async_copy(table.at[itl], blv, gsem).wait()
            pltpu.make_async_copy(table.at[itl], brv, gsem).wait()

            @pl.when(t >= 1)
            def _():
                pltpu.make_async_copy(outv, out.at[pl.ds(0, K)], ssem).wait()

            combine(buf)
            pltpu.async_copy(outv, out.at[pl.ds(qoff + c * K, K)], ssem)

            @pl.when(c + 2 < CPW)
            def _():
                compute_and_fire(c + 2, buf)
        return carry

    lax.fori_loop(0, CPW // 2, pair_body, 0)

    pltpu.make_async_copy(out0, out.at[pl.ds(0, K)], ssem0).wait()
    pltpu.make_async_copy(out1, out.at[pl.ds(0, K)], ssem1).wait()


@functools.partial(jax.jit, donate_argnums=())
def _warp(table, fy, fx):
    mesh = plsc.VectorSubcoreMesh(
        core_axis_name="c", subcore_axis_name="s",
        num_cores=NC, num_subcores=NS)
    kc = jnp.float32
    ki = jnp.int32
    return pl.kernel(
        _warp_body,
        out_type=jax.ShapeDtypeStruct((N, CP), kc),
        mesh=mesh,
        scratch_types=[
            pltpu.VMEM((K,), kc),
            pltpu.VMEM((K,), kc),
            pltpu.VMEM((K,), kc),
            pltpu.VMEM((K,), kc),
            pltpu.VMEM((K,), kc),
            pltpu.VMEM((K,), kc),
            pltpu.VMEM((K,), ki),
            pltpu.VMEM((K,), ki),
            pltpu.VMEM((K,), ki),
            pltpu.VMEM((K,), ki),
            pltpu.VMEM((K,), ki),
            pltpu.VMEM((K,), ki),
            pltpu.VMEM((K,), ki),
            pltpu.VMEM((K,), ki),
            pltpu.VMEM((K, CP), kc),
            pltpu.VMEM((K, CP), kc),
            pltpu.VMEM((K, CP), kc),
            pltpu.VMEM((K, CP), kc),
            pltpu.VMEM((K, CP), kc),
            pltpu.VMEM((K, CP), kc),
            pltpu.VMEM((K, CP), kc),
            pltpu.VMEM((K, CP), kc),
            pltpu.VMEM((K, CP), kc),
            pltpu.VMEM((K, CP), kc),
            pltpu.SemaphoreType.DMA,
            pltpu.SemaphoreType.DMA,
            pltpu.SemaphoreType.DMA,
            pltpu.SemaphoreType.DMA,
        ],
        compiler_params=pltpu.CompilerParams(use_tc_tiling_on_sc=False),
    )(table, fy, fx)


def kernel(x, flow):
    xp = jnp.concatenate(
        [x, jnp.zeros((B, H, W, CP - C), jnp.float32)], axis=-1
    ).reshape(N, CP)
    fy = flow[..., 0].reshape(N)
    fx = flow[..., 1].reshape(N)
    outp = _warp(xp, fy, fx)
    return outp[:, :C].reshape(B, H, W, C)
